# baseline (speedup 1.0000x reference)
import jax
import jax.numpy as jnp
from jax import lax
from jax.experimental import pallas as pl
from jax.experimental.pallas import tpu as pltpu

IN_C = 128
HEADS = 2
RATIO = 16
NBATCH = 8
NDST = NBATCH * RATIO
HC = HEADS * IN_C
HR = HEADS * RATIO
HD = HEADS * NDST
NEG_SLOPE = 0.2
MASKPOS = 3e38
MFLOOR = -1e33


def _bipart_pool_kernel(x_ref, batch_ref, xcb_ref, wl_ref, bl_ref, wr_ref,
                        br_ref, attc_ref, biasT_ref, out_ref,
                        attbd_s, s2n_s, til2_s, rep2_s, m_s, den_s, numT_s):
    f32 = jnp.float32
    t = pl.program_id(0)
    ntiles = pl.num_programs(0)

    @pl.when(t == 0)
    def _build_selectors():
        attc = attc_ref[...]
        cf = lax.broadcasted_iota(jnp.int32, (HC, HR), 0)
        jj = lax.broadcasted_iota(jnp.int32, (HC, HR), 1)
        attbd_s[...] = jnp.where(jj // RATIO == cf // IN_C, attc, 0.0)
        cf2 = lax.broadcasted_iota(jnp.int32, (HC, RATIO * HR), 0)
        kk = lax.broadcasted_iota(jnp.int32, (HC, RATIO * HR), 1)
        s2n_s[...] = jnp.where(
            kk % HR == (cf2 // IN_C) * RATIO + kk // HR,
            attc * (-(1.0 - NEG_SLOPE)), 0.0)
        j2 = lax.broadcasted_iota(jnp.int32, (HR, HD), 0)
        d2 = lax.broadcasted_iota(jnp.int32, (HR, HD), 1)
        til2_s[...] = jnp.where(
            (d2 // NDST == j2 // RATIO) & (d2 % RATIO == j2 % RATIO), 1.0, 0.0)
        bb = lax.broadcasted_iota(jnp.int32, (NBATCH, HD), 0)
        d3 = lax.broadcasted_iota(jnp.int32, (NBATCH, HD), 1)
        rep2_s[...] = jnp.where(d3 % NDST // RATIO == bb, 1.0, 0.0)

    xs = x_ref[...]
    batch_t = batch_ref[...]
    xl = jnp.dot(xs, wl_ref[...], preferred_element_type=f32, precision=lax.Precision.DEFAULT) + bl_ref[...]
    xr = (jnp.dot(xcb_ref[...], wr_ref[...], preferred_element_type=f32, precision=lax.Precision.DEFAULT)
          + br_ref[...])

    onehot = (batch_t == lax.broadcasted_iota(jnp.int32, (1, NBATCH), 1)
              ).astype(f32)
    ohrep = jnp.dot(onehot, rep2_s[...], preferred_element_type=f32, precision=lax.Precision.DEFAULT)

    u = jnp.dot(xl, attbd_s[...], preferred_element_type=f32, precision=lax.Precision.DEFAULT)
    vr = jnp.dot(xr, attbd_s[...], preferred_element_type=f32, precision=lax.Precision.DEFAULT)
    ind = (lax.broadcasted_iota(jnp.int32, (RATIO, HR), 1) % RATIO ==
           lax.broadcasted_iota(jnp.int32, (RATIO, HR), 0)).astype(f32)
    v = jnp.sum(vr * ind, axis=0, keepdims=True)

    logc = u + v
    for r in range(RATIO):
        nz = jnp.minimum(xl + xr[r:r + 1, :], 0.0)
        logc = logc + jnp.dot(nz, s2n_s[:, r * HR:(r + 1) * HR],
                              preferred_element_type=f32, precision=lax.Precision.DEFAULT)
    l2 = (jnp.dot(logc, til2_s[...], preferred_element_type=f32, precision=lax.Precision.DEFAULT) * ohrep
          + (ohrep - 1.0) * MASKPOS)
    m_t = jnp.maximum(jnp.max(l2, axis=0, keepdims=True), MFLOOR)

    @pl.when(t == 0)
    def _():
        p = jnp.exp(l2 - m_t)
        m_s[...] = m_t
        den_s[...] = jnp.sum(p, axis=0, keepdims=True)
        numT_s[...] = lax.dot_general(xl, p, (((0,), (0,)), ((), ())),
                                      preferred_element_type=f32, precision=lax.Precision.DEFAULT)

    @pl.when(t > 0)
    def _():
        m_old = m_s[...]
        m_new = jnp.maximum(m_old, m_t)
        corr = jnp.exp(m_old - m_new)
        p = jnp.exp(l2 - m_new)
        m_s[...] = m_new
        den_s[...] = den_s[...] * corr + jnp.sum(p, axis=0, keepdims=True)
        numT_s[...] = (numT_s[...] * corr
                       + lax.dot_general(xl, p, (((0,), (0,)), ((), ())),
                                         preferred_element_type=f32, precision=lax.Precision.DEFAULT))

    @pl.when(t == ntiles - 1)
    def _finalize():
        acc = jnp.zeros((IN_C, NDST), f32)
        for h in range(HEADS):
            blk = numT_s[h * IN_C:(h + 1) * IN_C, h * NDST:(h + 1) * NDST]
            acc = acc + blk / (den_s[0:1, h * NDST:(h + 1) * NDST] + 1e-16)
        out_ref[...] = jnp.transpose(acc * (1.0 / HEADS) + biasT_ref[...])


def _run(xp, bp, xcent_base, W_l, b_l, W_r, b_r, att, bias, tile_n):
    ntiles = xp.shape[0] // tile_n
    return pl.pallas_call(
        _bipart_pool_kernel,
        grid=(ntiles,),
        in_specs=[
            pl.BlockSpec((tile_n, IN_C), lambda t: (t, 0)),
            pl.BlockSpec((tile_n, 1), lambda t: (t, 0)),
            pl.BlockSpec((RATIO, IN_C), lambda t: (0, 0)),
            pl.BlockSpec((IN_C, HC), lambda t: (0, 0)),
            pl.BlockSpec((1, HC), lambda t: (0, 0)),
            pl.BlockSpec((IN_C, HC), lambda t: (0, 0)),
            pl.BlockSpec((1, HC), lambda t: (0, 0)),
            pl.BlockSpec((HC, 1), lambda t: (0, 0)),
            pl.BlockSpec((IN_C, 1), lambda t: (0, 0)),
        ],
        out_specs=pl.BlockSpec((NDST, IN_C), lambda t: (0, 0)),
        out_shape=jax.ShapeDtypeStruct((NDST, IN_C), jnp.float32),
        scratch_shapes=[
            pltpu.VMEM((HC, HR), jnp.float32),
            pltpu.VMEM((HC, RATIO * HR), jnp.float32),
            pltpu.VMEM((HR, HD), jnp.float32),
            pltpu.VMEM((NBATCH, HD), jnp.float32),
            pltpu.VMEM((1, HD), jnp.float32),
            pltpu.VMEM((1, HD), jnp.float32),
            pltpu.VMEM((HC, HD), jnp.float32),
        ],
    )(xp, bp, xcent_base, W_l, b_l.reshape(1, HC), W_r, b_r.reshape(1, HC),
      att.reshape(HC, 1), bias.reshape(IN_C, 1))


def kernel(x, edge_index, batch, xcent_base, W_l, b_l, W_r, b_r, att, bias):
    del edge_index
    n = x.shape[0]
    if n % 2000 == 0:
        out = _run(x, batch.astype(jnp.int32).reshape(n, 1), xcent_base,
                   W_l, b_l, W_r, b_r, att, bias, 2000)
    else:
        n_pad = -(-n // 1024) * 1024
        xp = jnp.pad(x, ((0, n_pad - n), (0, 0)))
        bp = jnp.pad(batch.astype(jnp.int32), (0, n_pad - n),
                     constant_values=NBATCH).reshape(n_pad, 1)
        out = _run(xp, bp, xcent_base, W_l, b_l, W_r, b_r, att, bias, 1024)
    return out.reshape(NBATCH, RATIO, IN_C)


# speedup vs baseline: 1.0121x; 1.0121x over previous
import jax
import jax.numpy as jnp
from jax import lax
from jax.experimental import pallas as pl
from jax.experimental.pallas import tpu as pltpu

IN_C = 128
HEADS = 2
RATIO = 16
NBATCH = 8
NDST = NBATCH * RATIO
HC = HEADS * IN_C
HR = HEADS * RATIO
HD = HEADS * NDST
NEG_SLOPE = 0.2
MASKPOS = 3e38
MFLOOR = -1e33


def _bipart_pool_kernel(x_ref, batch_ref, xcb_ref, wl_ref, bl_ref, wr_ref,
                        br_ref, attc_ref, biasT_ref, out_ref,
                        attbd_s, s2n_s, til2_s, rep2_s, m_s, den_s, numT_s):
    f32 = jnp.float32
    t = pl.program_id(0)
    ntiles = pl.num_programs(0)

    @pl.when(t == 0)
    def _build_selectors():
        attc = attc_ref[...]
        cf = lax.broadcasted_iota(jnp.int32, (HC, HR), 0)
        jj = lax.broadcasted_iota(jnp.int32, (HC, HR), 1)
        attbd_s[...] = jnp.where(jj // RATIO == cf // IN_C, attc, 0.0)
        cf2 = lax.broadcasted_iota(jnp.int32, (HC, RATIO * HR), 0)
        kk = lax.broadcasted_iota(jnp.int32, (HC, RATIO * HR), 1)
        s2n_s[...] = jnp.where(
            kk % HR == (cf2 // IN_C) * RATIO + kk // HR,
            attc * (-(1.0 - NEG_SLOPE)), 0.0).astype(jnp.bfloat16)
        j2 = lax.broadcasted_iota(jnp.int32, (HR, HD), 0)
        d2 = lax.broadcasted_iota(jnp.int32, (HR, HD), 1)
        til2_s[...] = jnp.where(
            (d2 // NDST == j2 // RATIO) & (d2 % RATIO == j2 % RATIO), 1.0, 0.0)
        bb = lax.broadcasted_iota(jnp.int32, (NBATCH, HD), 0)
        d3 = lax.broadcasted_iota(jnp.int32, (NBATCH, HD), 1)
        rep2_s[...] = jnp.where(d3 % NDST // RATIO == bb, 1.0, 0.0)

    xs = x_ref[...]
    batch_t = batch_ref[...]
    xl = jnp.dot(xs, wl_ref[...], preferred_element_type=f32) + bl_ref[...]
    xr = (jnp.dot(xcb_ref[...], wr_ref[...], preferred_element_type=f32)
          + br_ref[...])

    onehot = (batch_t == lax.broadcasted_iota(jnp.int32, (1, NBATCH), 1)
              ).astype(f32)
    ohrep = jnp.dot(onehot, rep2_s[...], preferred_element_type=f32)

    u = jnp.dot(xl, attbd_s[...], preferred_element_type=f32)
    vr = jnp.dot(xr, attbd_s[...], preferred_element_type=f32)
    ind = (lax.broadcasted_iota(jnp.int32, (RATIO, HR), 1) % RATIO ==
           lax.broadcasted_iota(jnp.int32, (RATIO, HR), 0)).astype(f32)
    v = jnp.sum(vr * ind, axis=0, keepdims=True)

    xl_bf = xl.astype(jnp.bfloat16)
    xr_bf = xr.astype(jnp.bfloat16)
    logc = u + v
    for r in range(RATIO):
        nz = jnp.minimum(xl_bf + xr_bf[r:r + 1, :], 0)
        logc = logc + jnp.dot(nz, s2n_s[:, r * HR:(r + 1) * HR],
                              preferred_element_type=f32)
    l2 = (jnp.dot(logc, til2_s[...], preferred_element_type=f32) * ohrep
          + (ohrep - 1.0) * MASKPOS)
    m_t = jnp.maximum(jnp.max(l2, axis=0, keepdims=True), MFLOOR)

    @pl.when(t == 0)
    def _():
        p = jnp.exp(l2 - m_t)
        m_s[...] = m_t
        den_s[...] = jnp.sum(p, axis=0, keepdims=True)
        numT_s[...] = lax.dot_general(xl_bf, p.astype(jnp.bfloat16),
                                      (((0,), (0,)), ((), ())),
                                      preferred_element_type=f32)

    @pl.when(t > 0)
    def _():
        m_old = m_s[...]
        m_new = jnp.maximum(m_old, m_t)
        corr = jnp.exp(m_old - m_new)
        p = jnp.exp(l2 - m_new)
        m_s[...] = m_new
        den_s[...] = den_s[...] * corr + jnp.sum(p, axis=0, keepdims=True)
        numT_s[...] = (numT_s[...] * corr
                       + lax.dot_general(xl_bf, p.astype(jnp.bfloat16),
                                         (((0,), (0,)), ((), ())),
                                         preferred_element_type=f32))

    @pl.when(t == ntiles - 1)
    def _finalize():
        acc = jnp.zeros((IN_C, NDST), f32)
        for h in range(HEADS):
            blk = numT_s[h * IN_C:(h + 1) * IN_C, h * NDST:(h + 1) * NDST]
            acc = acc + blk / (den_s[0:1, h * NDST:(h + 1) * NDST] + 1e-16)
        out_ref[...] = jnp.transpose(acc * (1.0 / HEADS) + biasT_ref[...])


def _run(xp, bp, xcent_base, W_l, b_l, W_r, b_r, att, bias, tile_n):
    ntiles = xp.shape[0] // tile_n
    return pl.pallas_call(
        _bipart_pool_kernel,
        grid=(ntiles,),
        in_specs=[
            pl.BlockSpec((tile_n, IN_C), lambda t: (t, 0)),
            pl.BlockSpec((tile_n, 1), lambda t: (t, 0)),
            pl.BlockSpec((RATIO, IN_C), lambda t: (0, 0)),
            pl.BlockSpec((IN_C, HC), lambda t: (0, 0)),
            pl.BlockSpec((1, HC), lambda t: (0, 0)),
            pl.BlockSpec((IN_C, HC), lambda t: (0, 0)),
            pl.BlockSpec((1, HC), lambda t: (0, 0)),
            pl.BlockSpec((HC, 1), lambda t: (0, 0)),
            pl.BlockSpec((IN_C, 1), lambda t: (0, 0)),
        ],
        out_specs=pl.BlockSpec((NDST, IN_C), lambda t: (0, 0)),
        out_shape=jax.ShapeDtypeStruct((NDST, IN_C), jnp.float32),
        scratch_shapes=[
            pltpu.VMEM((HC, HR), jnp.float32),
            pltpu.VMEM((HC, RATIO * HR), jnp.bfloat16),
            pltpu.VMEM((HR, HD), jnp.float32),
            pltpu.VMEM((NBATCH, HD), jnp.float32),
            pltpu.VMEM((1, HD), jnp.float32),
            pltpu.VMEM((1, HD), jnp.float32),
            pltpu.VMEM((HC, HD), jnp.float32),
        ],
    )(xp, bp, xcent_base, W_l, b_l.reshape(1, HC), W_r, b_r.reshape(1, HC),
      att.reshape(HC, 1), bias.reshape(IN_C, 1))


def kernel(x, edge_index, batch, xcent_base, W_l, b_l, W_r, b_r, att, bias):
    del edge_index
    n = x.shape[0]
    if n % 2000 == 0:
        out = _run(x, batch.astype(jnp.int32).reshape(n, 1), xcent_base,
                   W_l, b_l, W_r, b_r, att, bias, 2000)
    else:
        n_pad = -(-n // 1024) * 1024
        xp = jnp.pad(x, ((0, n_pad - n), (0, 0)))
        bp = jnp.pad(batch.astype(jnp.int32), (0, n_pad - n),
                     constant_values=NBATCH).reshape(n_pad, 1)
        out = _run(xp, bp, xcent_base, W_l, b_l, W_r, b_r, att, bias, 1024)
    return out.reshape(NBATCH, RATIO, IN_C)


# speedup vs baseline: 1.0378x; 1.0253x over previous
import jax
import jax.numpy as jnp
from jax import lax
from jax.experimental import pallas as pl
from jax.experimental.pallas import tpu as pltpu

IN_C = 128
HEADS = 2
RATIO = 16
NBATCH = 8
NDST = NBATCH * RATIO
HC = HEADS * IN_C
HR = HEADS * RATIO
HD = HEADS * NDST
NEG_SLOPE = 0.2
MASKPOS = 3e38
MFLOOR = -1e33


def _bipart_pool_kernel(x_ref, batch_ref, xcb_ref, wl_ref, bl_ref, wr_ref,
                        br_ref, attc_ref, biasT_ref, out_ref,
                        attbd_s, s2n_s, til2_s, rep2_s, m_s, den_s, numT_s):
    f32 = jnp.float32
    t = pl.program_id(0)
    ntiles = pl.num_programs(0)

    @pl.when(t == 0)
    def _build_selectors():
        attc = attc_ref[...]
        cf = lax.broadcasted_iota(jnp.int32, (HC, HR), 0)
        jj = lax.broadcasted_iota(jnp.int32, (HC, HR), 1)
        attbd_s[...] = jnp.where(jj // RATIO == cf // IN_C, attc,
                                 0.0).astype(jnp.bfloat16)
        cf2 = lax.broadcasted_iota(jnp.int32, (HC, RATIO * HR), 0)
        kk = lax.broadcasted_iota(jnp.int32, (HC, RATIO * HR), 1)
        s2n_s[...] = jnp.where(
            kk % HR == (cf2 // IN_C) * RATIO + kk // HR,
            attc * (-(1.0 - NEG_SLOPE)), 0.0).astype(jnp.bfloat16)
        j2 = lax.broadcasted_iota(jnp.int32, (HR, HD), 0)
        d2 = lax.broadcasted_iota(jnp.int32, (HR, HD), 1)
        til2_s[...] = jnp.where(
            (d2 // NDST == j2 // RATIO) & (d2 % RATIO == j2 % RATIO),
            1.0, 0.0).astype(jnp.bfloat16)
        bb = lax.broadcasted_iota(jnp.int32, (NBATCH + 1, HD), 0)
        d3 = lax.broadcasted_iota(jnp.int32, (NBATCH + 1, HD), 1)
        rep2_s[...] = jnp.where(d3 % NDST // RATIO == bb, 0.0,
                                -MASKPOS).astype(jnp.bfloat16)

    xs = x_ref[...]
    batch_t = batch_ref[...]
    xl = jnp.dot(xs, wl_ref[...], preferred_element_type=f32) + bl_ref[...]
    xr = (jnp.dot(xcb_ref[...], wr_ref[...], preferred_element_type=f32)
          + br_ref[...])

    onehot = (batch_t == lax.broadcasted_iota(jnp.int32, (1, NBATCH + 1), 1)
              ).astype(jnp.bfloat16)
    maskadd = jnp.dot(onehot, rep2_s[...], preferred_element_type=f32)

    xl_bf = xl.astype(jnp.bfloat16)
    xr_bf = xr.astype(jnp.bfloat16)

    u = jnp.dot(xl_bf, attbd_s[...], preferred_element_type=f32)
    vr = jnp.dot(xr_bf, attbd_s[...], preferred_element_type=f32)
    ind = (lax.broadcasted_iota(jnp.int32, (RATIO, HR), 1) % RATIO ==
           lax.broadcasted_iota(jnp.int32, (RATIO, HR), 0)).astype(f32)
    v = jnp.sum(vr * ind, axis=0, keepdims=True)

    logc = u + v
    for r in range(RATIO):
        nz = jnp.minimum(xl_bf + xr_bf[r:r + 1, :], 0)
        logc = logc + jnp.dot(nz, s2n_s[:, r * HR:(r + 1) * HR],
                              preferred_element_type=f32)
    l2 = (jnp.dot(logc.astype(jnp.bfloat16), til2_s[...],
                  preferred_element_type=f32) + maskadd)
    m_t = jnp.maximum(jnp.max(l2, axis=0, keepdims=True), MFLOOR)

    @pl.when(t == 0)
    def _():
        p = jnp.exp(l2 - m_t)
        m_s[...] = m_t
        den_s[...] = jnp.sum(p, axis=0, keepdims=True)
        numT_s[...] = lax.dot_general(xl_bf, p.astype(jnp.bfloat16),
                                      (((0,), (0,)), ((), ())),
                                      preferred_element_type=f32)

    @pl.when(t > 0)
    def _():
        m_old = m_s[...]
        m_new = jnp.maximum(m_old, m_t)
        corr = jnp.exp(m_old - m_new)
        p = jnp.exp(l2 - m_new)
        m_s[...] = m_new
        den_s[...] = den_s[...] * corr + jnp.sum(p, axis=0, keepdims=True)
        numT_s[...] = (numT_s[...] * corr
                       + lax.dot_general(xl_bf, p.astype(jnp.bfloat16),
                                         (((0,), (0,)), ((), ())),
                                         preferred_element_type=f32))

    @pl.when(t == ntiles - 1)
    def _finalize():
        acc = jnp.zeros((IN_C, NDST), f32)
        for h in range(HEADS):
            blk = numT_s[h * IN_C:(h + 1) * IN_C, h * NDST:(h + 1) * NDST]
            acc = acc + blk / (den_s[0:1, h * NDST:(h + 1) * NDST] + 1e-16)
        out_ref[...] = jnp.transpose(acc * (1.0 / HEADS) + biasT_ref[...])


def _run(xp, bp, xcent_base, W_l, b_l, W_r, b_r, att, bias, tile_n):
    ntiles = xp.shape[0] // tile_n
    return pl.pallas_call(
        _bipart_pool_kernel,
        grid=(ntiles,),
        in_specs=[
            pl.BlockSpec((tile_n, IN_C), lambda t: (t, 0)),
            pl.BlockSpec((tile_n, 1), lambda t: (t, 0)),
            pl.BlockSpec((RATIO, IN_C), lambda t: (0, 0)),
            pl.BlockSpec((IN_C, HC), lambda t: (0, 0)),
            pl.BlockSpec((1, HC), lambda t: (0, 0)),
            pl.BlockSpec((IN_C, HC), lambda t: (0, 0)),
            pl.BlockSpec((1, HC), lambda t: (0, 0)),
            pl.BlockSpec((HC, 1), lambda t: (0, 0)),
            pl.BlockSpec((IN_C, 1), lambda t: (0, 0)),
        ],
        out_specs=pl.BlockSpec((NDST, IN_C), lambda t: (0, 0)),
        out_shape=jax.ShapeDtypeStruct((NDST, IN_C), jnp.float32),
        scratch_shapes=[
            pltpu.VMEM((HC, HR), jnp.bfloat16),
            pltpu.VMEM((HC, RATIO * HR), jnp.bfloat16),
            pltpu.VMEM((HR, HD), jnp.bfloat16),
            pltpu.VMEM((NBATCH + 1, HD), jnp.bfloat16),
            pltpu.VMEM((1, HD), jnp.float32),
            pltpu.VMEM((1, HD), jnp.float32),
            pltpu.VMEM((HC, HD), jnp.float32),
        ],
    )(xp, bp, xcent_base, W_l, b_l.reshape(1, HC), W_r, b_r.reshape(1, HC),
      att.reshape(HC, 1), bias.reshape(IN_C, 1))


def kernel(x, edge_index, batch, xcent_base, W_l, b_l, W_r, b_r, att, bias):
    del edge_index
    n = x.shape[0]
    if n % 5000 == 0:
        out = _run(x, batch.astype(jnp.int32).reshape(n, 1), xcent_base,
                   W_l, b_l, W_r, b_r, att, bias, 5000)
    elif n % 2000 == 0:
        out = _run(x, batch.astype(jnp.int32).reshape(n, 1), xcent_base,
                   W_l, b_l, W_r, b_r, att, bias, 2000)
    else:
        n_pad = -(-n // 1024) * 1024
        xp = jnp.pad(x, ((0, n_pad - n), (0, 0)))
        bp = jnp.pad(batch.astype(jnp.int32), (0, n_pad - n),
                     constant_values=NBATCH).reshape(n_pad, 1)
        out = _run(xp, bp, xcent_base, W_l, b_l, W_r, b_r, att, bias, 1024)
    return out.reshape(NBATCH, RATIO, IN_C)


# speedup vs baseline: 1.0531x; 1.0148x over previous
import jax
import jax.numpy as jnp
from jax import lax
from jax.experimental import pallas as pl
from jax.experimental.pallas import tpu as pltpu

IN_C = 128
HEADS = 2
RATIO = 16
NBATCH = 8
NDST = NBATCH * RATIO
HC = HEADS * IN_C
HR = HEADS * RATIO
HD = HEADS * NDST
NEG_SLOPE = 0.2


def _bipart_pool_kernel(x_ref, batch_ref, xcb_ref, wl_ref, bl_ref, wr_ref,
                        br_ref, attc_ref, biasT_ref, out_ref,
                        attbd_s, s2n_s, til2_s, rep2_s, m_s, den_s, numT_s):
    f32 = jnp.float32
    t = pl.program_id(0)
    ntiles = pl.num_programs(0)

    @pl.when(t == 0)
    def _build_selectors():
        attc = attc_ref[...]
        cf = lax.broadcasted_iota(jnp.int32, (HC, HR), 0)
        jj = lax.broadcasted_iota(jnp.int32, (HC, HR), 1)
        attbd_s[...] = jnp.where(jj // RATIO == cf // IN_C, attc,
                                 0.0).astype(jnp.bfloat16)
        cf2 = lax.broadcasted_iota(jnp.int32, (HC, RATIO * HR), 0)
        kk = lax.broadcasted_iota(jnp.int32, (HC, RATIO * HR), 1)
        s2n_s[...] = jnp.where(
            kk % HR == (cf2 // IN_C) * RATIO + kk // HR,
            attc * (-(1.0 - NEG_SLOPE)), 0.0).astype(jnp.bfloat16)
        j2 = lax.broadcasted_iota(jnp.int32, (HR, HD), 0)
        d2 = lax.broadcasted_iota(jnp.int32, (HR, HD), 1)
        til2_s[...] = jnp.where(
            (d2 // NDST == j2 // RATIO) & (d2 % RATIO == j2 % RATIO),
            1.0, 0.0).astype(jnp.bfloat16)
        bb = lax.broadcasted_iota(jnp.int32, (NBATCH + 1, HD), 0)
        d3 = lax.broadcasted_iota(jnp.int32, (NBATCH + 1, HD), 1)
        rep2_s[...] = jnp.where(d3 % NDST // RATIO == bb, 1.0,
                                0.0).astype(jnp.bfloat16)

    xs = x_ref[...]
    batch_t = batch_ref[...]
    xl = jnp.dot(xs, wl_ref[...], preferred_element_type=f32) + bl_ref[...]
    xr = (jnp.dot(xcb_ref[...], wr_ref[...], preferred_element_type=f32)
          + br_ref[...])

    bcol = lax.broadcasted_iota(jnp.int32, (1, HD), 1) % NDST // RATIO
    ohm = (batch_t == bcol).astype(jnp.bfloat16)

    xl_bf = xl.astype(jnp.bfloat16)
    xr_bf = xr.astype(jnp.bfloat16)

    u = jnp.dot(xl_bf, attbd_s[...], preferred_element_type=f32)
    vr = jnp.dot(xr_bf, attbd_s[...], preferred_element_type=f32)
    ind = (lax.broadcasted_iota(jnp.int32, (RATIO, HR), 1) % RATIO ==
           lax.broadcasted_iota(jnp.int32, (RATIO, HR), 0)).astype(f32)
    v = jnp.sum(vr * ind, axis=0, keepdims=True)

    logc_a = u + v
    logc_b = jnp.zeros_like(logc_a)
    for r in range(0, RATIO, 2):
        nz = jnp.minimum(xl_bf + xr_bf[r:r + 1, :], 0)
        logc_a = logc_a + jnp.dot(nz, s2n_s[:, r * HR:(r + 1) * HR],
                                  preferred_element_type=f32)
        nz2 = jnp.minimum(xl_bf + xr_bf[r + 1:r + 2, :], 0)
        logc_b = logc_b + jnp.dot(nz2, s2n_s[:, (r + 1) * HR:(r + 2) * HR],
                                  preferred_element_type=f32)
    logc = logc_a + logc_b

    m_t = jnp.max(logc, axis=0, keepdims=True)

    @pl.when(t == 0)
    def _():
        pc = jnp.exp(logc - m_t).astype(jnp.bfloat16)
        pm = jnp.dot(pc, til2_s[...],
                     preferred_element_type=f32).astype(jnp.bfloat16) * ohm
        m_s[...] = m_t
        den_s[...] = jnp.sum(pm, axis=0, keepdims=True, dtype=f32)
        numT_s[...] = lax.dot_general(xl_bf, pm, (((0,), (0,)), ((), ())),
                                      preferred_element_type=f32)

    @pl.when(t > 0)
    def _():
        m_old = m_s[...]
        m_new = jnp.maximum(m_old, m_t)
        corr_bf = jnp.exp(m_old - m_new).astype(jnp.bfloat16)
        corrfull = jnp.dot(corr_bf, til2_s[...],
                           preferred_element_type=f32)
        pc = jnp.exp(logc - m_new).astype(jnp.bfloat16)
        pm = jnp.dot(pc, til2_s[...],
                     preferred_element_type=f32).astype(jnp.bfloat16) * ohm
        m_s[...] = m_new
        den_s[...] = (den_s[...] * corrfull
                      + jnp.sum(pm, axis=0, keepdims=True, dtype=f32))
        numT_s[...] = (numT_s[...] * corrfull
                       + lax.dot_general(xl_bf, pm, (((0,), (0,)), ((), ())),
                                         preferred_element_type=f32))

    @pl.when(t == ntiles - 1)
    def _finalize():
        acc = jnp.zeros((IN_C, NDST), f32)
        for h in range(HEADS):
            blk = numT_s[h * IN_C:(h + 1) * IN_C, h * NDST:(h + 1) * NDST]
            acc = acc + blk / (den_s[0:1, h * NDST:(h + 1) * NDST] + 1e-16)
        out_ref[...] = jnp.transpose(acc * (1.0 / HEADS) + biasT_ref[...])


def _run(xp, bp, xcent_base, W_l, b_l, W_r, b_r, att, bias, tile_n):
    ntiles = xp.shape[0] // tile_n
    return pl.pallas_call(
        _bipart_pool_kernel,
        grid=(ntiles,),
        in_specs=[
            pl.BlockSpec((tile_n, IN_C), lambda t: (t, 0)),
            pl.BlockSpec((tile_n, 1), lambda t: (t, 0)),
            pl.BlockSpec((RATIO, IN_C), lambda t: (0, 0)),
            pl.BlockSpec((IN_C, HC), lambda t: (0, 0)),
            pl.BlockSpec((1, HC), lambda t: (0, 0)),
            pl.BlockSpec((IN_C, HC), lambda t: (0, 0)),
            pl.BlockSpec((1, HC), lambda t: (0, 0)),
            pl.BlockSpec((HC, 1), lambda t: (0, 0)),
            pl.BlockSpec((IN_C, 1), lambda t: (0, 0)),
        ],
        out_specs=pl.BlockSpec((NDST, IN_C), lambda t: (0, 0)),
        out_shape=jax.ShapeDtypeStruct((NDST, IN_C), jnp.float32),
        scratch_shapes=[
            pltpu.VMEM((HC, HR), jnp.bfloat16),
            pltpu.VMEM((HC, RATIO * HR), jnp.bfloat16),
            pltpu.VMEM((HR, HD), jnp.bfloat16),
            pltpu.VMEM((NBATCH + 1, HD), jnp.bfloat16),
            pltpu.VMEM((1, HR), jnp.float32),
            pltpu.VMEM((1, HD), jnp.float32),
            pltpu.VMEM((HC, HD), jnp.float32),
        ],
    )(xp, bp, xcent_base, W_l, b_l.reshape(1, HC), W_r, b_r.reshape(1, HC),
      att.reshape(HC, 1), bias.reshape(IN_C, 1))


def kernel(x, edge_index, batch, xcent_base, W_l, b_l, W_r, b_r, att, bias):
    del edge_index
    n = x.shape[0]
    if n % 5000 == 0:
        out = _run(x, batch.astype(jnp.int32).reshape(n, 1), xcent_base,
                   W_l, b_l, W_r, b_r, att, bias, 5000)
    elif n % 2000 == 0:
        out = _run(x, batch.astype(jnp.int32).reshape(n, 1), xcent_base,
                   W_l, b_l, W_r, b_r, att, bias, 2000)
    else:
        n_pad = -(-n // 1024) * 1024
        xp = jnp.pad(x, ((0, n_pad - n), (0, 0)))
        bp = jnp.pad(batch.astype(jnp.int32), (0, n_pad - n),
                     constant_values=NBATCH).reshape(n_pad, 1)
        out = _run(xp, bp, xcent_base, W_l, b_l, W_r, b_r, att, bias, 1024)
    return out.reshape(NBATCH, RATIO, IN_C)
